# baseline (speedup 1.0000x reference)
import functools

import jax
import jax.numpy as jnp
from jax import lax
from jax.experimental import pallas as pl
from jax.experimental.pallas import tpu as pltpu
from jax.experimental.pallas import tpu_sc as plsc

MODEL_DIM = 128
SCALE = float(MODEL_DIM) ** 0.5

NUM_CORES = 2
NUM_SUBCORES = 16
LANES = 16
NUM_WORKERS = NUM_CORES * NUM_SUBCORES

CHUNK = 128
NBUF = 4
SCALE_BLK = 2000


def _prescale_body(t_ref, o_ref):
  o_ref[...] = t_ref[...] * SCALE


def _prescale(table):
  v, d = table.shape
  return pl.pallas_call(
      _prescale_body,
      out_shape=jax.ShapeDtypeStruct((v, d), table.dtype),
      grid=(v // SCALE_BLK,),
      in_specs=[pl.BlockSpec((SCALE_BLK, d), lambda i: (i, 0))],
      out_specs=pl.BlockSpec((SCALE_BLK, d), lambda i: (i, 0)),
  )(table)


@functools.partial(jax.jit, static_argnames=("n_rows",))
def _gather(idx2d, table, n_rows):
  d = table.shape[1]
  n_chunks = idx2d.shape[0]
  ch_per_w = n_chunks // NUM_WORKERS

  mesh = plsc.VectorSubcoreMesh(core_axis_name="c", subcore_axis_name="s")

  @functools.partial(
      pl.kernel,
      mesh=mesh,
      out_type=jax.ShapeDtypeStruct((n_rows, d), jnp.float32),
      scratch_types=[
          pltpu.VMEM((ch_per_w, CHUNK), jnp.int32),
          pltpu.VMEM((NBUF, CHUNK, d), jnp.float32),
      ] + [pltpu.SemaphoreType.DMA] * (2 * NBUF),
  )
  def k(table_hbm, idx_hbm, out_hbm, idx_v, rows, *sems):
    gsems = sems[:NBUF]
    ssems = sems[NBUF:]
    wid = lax.axis_index("s") * NUM_CORES + lax.axis_index("c")
    ch_base = wid * ch_per_w
    bufs = [rows.at[b] for b in range(NBUF)]

    pltpu.sync_copy(idx_hbm.at[pl.ds(ch_base, ch_per_w)], idx_v)

    def gather(i, b):
      pltpu.async_copy(table_hbm.at[idx_v.at[i]], bufs[b], gsems[b])

    def wait_gather(b):
      pltpu.make_async_copy(table_hbm.at[idx_v.at[0]], bufs[b],
                            gsems[b]).wait()

    def store(i, b):
      pltpu.async_copy(bufs[b],
                       out_hbm.at[pl.ds((ch_base + i) * CHUNK, CHUNK)],
                       ssems[b])

    def wait_store(b):
      pltpu.make_async_copy(bufs[b], out_hbm.at[pl.ds(0, CHUNK)],
                            ssems[b]).wait()

    for b in range(NBUF):
      gather(b, b)

    def ring_body(g, _):
      for b in range(NBUF):
        i = NBUF * g + b

        wait_gather(b)
        store(i, b)

        @pl.when(i + NBUF < ch_per_w)
        def _():
          wait_store(b)
          gather(i + NBUF, b)
      return 0

    lax.fori_loop(0, ch_per_w // NBUF, ring_body, 0)
    for b in range(NBUF):
      wait_store(b)

  return k(table, idx2d)


def kernel(inputs, embeddings):
  b, t = inputs.shape
  n_rows = b * t
  idx2d = inputs.reshape(n_rows // CHUNK, CHUNK).astype(jnp.int32)
  scaled = _prescale(embeddings)
  out = _gather(idx2d, scaled, n_rows)
  return out.reshape(b, t, embeddings.shape[1])


# speedup vs baseline: 1.1520x; 1.1520x over previous
import functools

import jax
import jax.numpy as jnp
from jax import lax
from jax.experimental import pallas as pl
from jax.experimental.pallas import tpu as pltpu
from jax.experimental.pallas import tpu_sc as plsc

MODEL_DIM = 128
SCALE = float(MODEL_DIM) ** 0.5

NUM_CORES = 2
NUM_SUBCORES = 16
LANES = 16
NUM_WORKERS = NUM_CORES * NUM_SUBCORES

CHUNK = 128
NBUF = 5
PREF = 3


@functools.partial(jax.jit, static_argnames=("n_rows",))
def _gather_scale(idx2d, table, n_rows):
  d = table.shape[1]
  n_chunks = idx2d.shape[0]
  ch_per_w = n_chunks // NUM_WORKERS

  mesh = plsc.VectorSubcoreMesh(core_axis_name="c", subcore_axis_name="s")

  @functools.partial(
      pl.kernel,
      mesh=mesh,
      out_type=jax.ShapeDtypeStruct((n_rows, d), jnp.float32),
      scratch_types=[
          pltpu.VMEM((ch_per_w, CHUNK), jnp.int32),
          pltpu.VMEM((NBUF, CHUNK, d), jnp.float32),
      ] + [pltpu.SemaphoreType.DMA] * (2 * NBUF),
  )
  def k(table_hbm, idx_hbm, out_hbm, idx_v, rows, *sems):
    gsems = sems[:NBUF]
    ssems = sems[NBUF:]
    wid = lax.axis_index("s") * NUM_CORES + lax.axis_index("c")
    ch_base = wid * ch_per_w
    bufs = [rows.at[b] for b in range(NBUF)]

    pltpu.sync_copy(idx_hbm.at[pl.ds(ch_base, ch_per_w)], idx_v)

    def gather(i, b):
      pltpu.async_copy(table_hbm.at[idx_v.at[i]], bufs[b], gsems[b])

    def wait_gather(b):
      pltpu.make_async_copy(table_hbm.at[idx_v.at[0]], bufs[b],
                            gsems[b]).wait()

    def store(i, b):
      pltpu.async_copy(bufs[b],
                       out_hbm.at[pl.ds((ch_base + i) * CHUNK, CHUNK)],
                       ssems[b])

    def wait_store(b):
      pltpu.make_async_copy(bufs[b], out_hbm.at[pl.ds(0, CHUNK)],
                            ssems[b]).wait()

    for i in range(PREF):
      gather(i, i)

    def ring_body(g, _):
      for b in range(NBUF):
        i = NBUF * g + b
        f = i + PREF
        bf = (b + PREF) % NBUF

        @pl.when(f < ch_per_w)
        def _():
          @pl.when(f >= NBUF)
          def _():
            wait_store(bf)
          gather(f, bf)

        wait_gather(b)

        def row_body(r, _):
          for t in range(d // LANES):
            sl = pl.ds(t * LANES, LANES)
            rows[b, r, sl] = rows[b, r, sl] * SCALE
          return 0

        lax.fori_loop(0, CHUNK, row_body, 0, unroll=2)
        store(i, b)
      return 0

    lax.fori_loop(0, ch_per_w // NBUF, ring_body, 0)
    for b in range(NBUF):
      wait_store(b)

  return k(table, idx2d)


def kernel(inputs, embeddings):
  b, t = inputs.shape
  n_rows = b * t
  idx2d = inputs.reshape(n_rows // CHUNK, CHUNK).astype(jnp.int32)
  out = _gather_scale(idx2d, embeddings, n_rows)
  return out.reshape(b, t, embeddings.shape[1])
